# Initial kernel scaffold; baseline (speedup 1.0000x reference)
import jax
import jax.numpy as jnp
from jax.experimental import pallas as pl


def kernel(x, B, Pi):
    raise NotImplementedError("write your pallas kernel here")


# speedup vs baseline: 3.2916x; 3.2916x over previous
import functools

import jax
import jax.numpy as jnp
from jax import lax
from jax.experimental import pallas as pl
from jax.experimental.pallas import tpu as pltpu
from jax.experimental.pallas import tpu_sc as plsc

_C = 32
_M = 128
_NG = 16
_D = _C * _NG
_N = 100000
_CH = 80
_NCH = _N // _CH
_NW = 32


def _table_body(b_ref, pi_ref, post_ref, ll_ref):
    b = b_ref[...]
    pi = pi_ref[...][:1, :]
    r0 = lax.broadcasted_iota(jnp.int32, (_D, _D), 0)
    r1 = lax.broadcasted_iota(jnp.int32, (_D, _D), 1)
    k_mat = ((r0 % _NG) == (r1 % _NG)).astype(jnp.float32)

    bmax = jnp.max(b, axis=0, keepdims=True)
    be = jnp.exp(b - bmax)
    sm_b = be / jnp.sum(be, axis=0, keepdims=True)

    pe = jnp.exp(pi)
    pz = jnp.dot(pe, k_mat, preferred_element_type=jnp.float32)
    sm_pi = pe / pz

    un = sm_pi * sm_b + 1e-8
    s_b = jnp.dot(un, k_mat, preferred_element_type=jnp.float32)
    post_ref[...] = un / s_b
    ll_ref[...] = jnp.log(s_b[:, :_M])


def _build_tables(b2d, pi2d):
    return pl.pallas_call(
        _table_body,
        out_shape=(
            jax.ShapeDtypeStruct((_M, _D), jnp.float32),
            jax.ShapeDtypeStruct((_M, _M), jnp.float32),
        ),
    )(b2d, pi2d)


def _gather_body(tab_hbm, llt_hbm, x_hbm, post_out, ll_out,
                 idx_v, rows_v, llrows_v, sem1, sem2):
    wid = lax.axis_index("s") * 2 + lax.axis_index("c")
    nt = (_NCH - wid + _NW - 1) // _NW

    def body(t, carry):
        base = (wid + t * _NW) * _CH
        pltpu.sync_copy(x_hbm.at[pl.ds(base, _CH)], idx_v)
        cp1 = pltpu.async_copy(tab_hbm.at[idx_v], rows_v, sem1)
        cp2 = pltpu.async_copy(llt_hbm.at[idx_v], llrows_v, sem2)
        cp1.wait()
        cp2.wait()
        pltpu.sync_copy(rows_v, post_out.at[pl.ds(base, _CH)])
        pltpu.sync_copy(llrows_v.at[:, pl.ds(0, _NG)],
                        ll_out.at[pl.ds(base, _CH)])
        return carry

    lax.fori_loop(0, nt, body, 0)


def _gather(tab, llt, xi):
    mesh = plsc.VectorSubcoreMesh(core_axis_name="c", subcore_axis_name="s")
    f = functools.partial(
        pl.kernel,
        mesh=mesh,
        compiler_params=pltpu.CompilerParams(use_tc_tiling_on_sc=False),
        out_type=(
            jax.ShapeDtypeStruct((_N, _D), jnp.float32),
            jax.ShapeDtypeStruct((_N, _NG), jnp.float32),
        ),
        scratch_types=[
            pltpu.VMEM((_CH,), jnp.int32),
            pltpu.VMEM((_CH, _D), jnp.float32),
            pltpu.VMEM((_CH, _M), jnp.float32),
            pltpu.SemaphoreType.DMA,
            pltpu.SemaphoreType.DMA,
        ],
    )(_gather_body)
    return f(tab, llt, xi)


def kernel(x, B, Pi):
    xi = x.astype(jnp.int32)
    b2d = jnp.transpose(B, (1, 0, 2)).reshape(_M, _D)
    pi2d = jnp.broadcast_to(Pi.reshape(1, _D), (8, _D))
    tab, llt = _build_tables(b2d, pi2d)
    post_flat, ll = _gather(tab, llt, xi)
    return ll, post_flat.reshape(_N, _C, _NG)


# speedup vs baseline: 4.3605x; 1.3247x over previous
import functools

import jax
import jax.numpy as jnp
from jax import lax
from jax.experimental import pallas as pl
from jax.experimental.pallas import tpu as pltpu
from jax.experimental.pallas import tpu_sc as plsc

_C = 32
_M = 128
_NG = 16
_D = _C * _NG
_N = 100000
_CH = 80
_NCH = _N // _CH
_NW = 32


def _table_body(b_ref, pi_ref, post_ref, ll_ref):
    b = b_ref[...]
    pi = pi_ref[...][:1, :]
    r0 = lax.broadcasted_iota(jnp.int32, (_D, _D), 0)
    r1 = lax.broadcasted_iota(jnp.int32, (_D, _D), 1)
    k_mat = ((r0 % _NG) == (r1 % _NG)).astype(jnp.float32)

    bmax = jnp.max(b, axis=0, keepdims=True)
    be = jnp.exp(b - bmax)
    sm_b = be / jnp.sum(be, axis=0, keepdims=True)

    pe = jnp.exp(pi)
    pz = jnp.dot(pe, k_mat, preferred_element_type=jnp.float32)
    sm_pi = pe / pz

    un = sm_pi * sm_b + 1e-8
    s_b = jnp.dot(un, k_mat, preferred_element_type=jnp.float32)
    post_ref[...] = un / s_b
    ll_ref[...] = jnp.log(s_b[:, :_M])


def _build_tables(b2d, pi2d):
    return pl.pallas_call(
        _table_body,
        out_shape=(
            jax.ShapeDtypeStruct((_M, _D), jnp.float32),
            jax.ShapeDtypeStruct((_M, _M), jnp.float32),
        ),
    )(b2d, pi2d)


def _gather_body(tab_hbm, llt_hbm, x_hbm, post_out, ll_out,
                 idx_v, rows_v, llrows_v, sem1, sem2):
    wid = lax.axis_index("s") * 2 + lax.axis_index("c")
    nt = (_NCH - wid + _NW - 1) // _NW

    def body(t, carry):
        base = (wid + t * _NW) * _CH
        pltpu.sync_copy(x_hbm.at[pl.ds(base, _CH)], idx_v)
        cp1 = pltpu.async_copy(tab_hbm.at[idx_v], rows_v, sem1)
        cp2 = pltpu.async_copy(llt_hbm.at[idx_v], llrows_v, sem2)
        cp1.wait()
        cp2.wait()
        pltpu.sync_copy(rows_v, post_out.at[pl.ds(base, _CH)])
        pltpu.sync_copy(llrows_v, ll_out.at[pl.ds(base, _CH)])
        return carry

    lax.fori_loop(0, nt, body, 0)


def _gather(tab, llt, xi):
    mesh = plsc.VectorSubcoreMesh(core_axis_name="c", subcore_axis_name="s")
    f = functools.partial(
        pl.kernel,
        mesh=mesh,
        out_type=(
            jax.ShapeDtypeStruct((_N, _D), jnp.float32),
            jax.ShapeDtypeStruct((_N, _M), jnp.float32),
        ),
        scratch_types=[
            pltpu.VMEM((_CH,), jnp.int32),
            pltpu.VMEM((_CH, _D), jnp.float32),
            pltpu.VMEM((_CH, _M), jnp.float32),
            pltpu.SemaphoreType.DMA,
            pltpu.SemaphoreType.DMA,
        ],
    )(_gather_body)
    return f(tab, llt, xi)


def kernel(x, B, Pi):
    xi = x.astype(jnp.int32)
    b2d = jnp.transpose(B, (1, 0, 2)).reshape(_M, _D)
    pi2d = jnp.broadcast_to(Pi.reshape(1, _D), (8, _D))
    tab, llt = _build_tables(b2d, pi2d)
    post_flat, ll_pad = _gather(tab, llt, xi)
    return ll_pad[:, :_NG], post_flat.reshape(_N, _C, _NG)


# speedup vs baseline: 4.7747x; 1.0950x over previous
import functools

import jax
import jax.numpy as jnp
from jax import lax
from jax.experimental import pallas as pl
from jax.experimental.pallas import tpu as pltpu
from jax.experimental.pallas import tpu_sc as plsc

_C = 32
_M = 128
_NG = 16
_D = _C * _NG
_N = 100000
_CH = 80
_NCH = _N // _CH
_NW = 32
_LBW = 2048
_NLB = (_N + _LBW - 1) // _LBW
_NPAD = _NLB * _LBW


def _table_body(b_ref, pi_ref, tab_ref, llt_ref):
    b = b_ref[...]
    pi = pi_ref[...][:1, :]
    r0 = lax.broadcasted_iota(jnp.int32, (_D, _D), 0)
    r1 = lax.broadcasted_iota(jnp.int32, (_D, _D), 1)
    k_mat = ((r0 % _NG) == (r1 % _NG)).astype(jnp.float32)

    bmax = jnp.max(b, axis=0, keepdims=True)
    be = jnp.exp(b - bmax)
    sm_b = be / jnp.sum(be, axis=0, keepdims=True)

    pe = jnp.exp(pi)
    pz = jnp.dot(pe, k_mat, preferred_element_type=jnp.float32)
    sm_pi = pe / pz

    un = sm_pi * sm_b + 1e-8
    s_b = jnp.dot(un, k_mat, preferred_element_type=jnp.float32)
    tab_ref[...] = un / s_b
    llt_ref[...] = jnp.transpose(jnp.log(s_b[:, :_NG]), (1, 0))


def _build_tables(b2d, pi2d):
    return pl.pallas_call(
        _table_body,
        out_shape=(
            jax.ShapeDtypeStruct((_M, _D), jnp.float32),
            jax.ShapeDtypeStruct((_NG, _M), jnp.float32),
        ),
    )(b2d, pi2d)


def _gather_body(tab_hbm, x_hbm, post_out, idx_v, rows_v, sem1):
    wid = lax.axis_index("s") * 2 + lax.axis_index("c")
    nt = (_NCH - wid + _NW - 1) // _NW

    def body(t, carry):
        base = (wid + t * _NW) * _CH
        pltpu.sync_copy(x_hbm.at[pl.ds(base, _CH)], idx_v)
        pltpu.async_copy(tab_hbm.at[idx_v], rows_v, sem1).wait()
        pltpu.sync_copy(rows_v, post_out.at[pl.ds(base, _CH)])
        return carry

    lax.fori_loop(0, nt, body, 0)


def _gather(tab, xi):
    mesh = plsc.VectorSubcoreMesh(core_axis_name="c", subcore_axis_name="s")
    f = functools.partial(
        pl.kernel,
        mesh=mesh,
        out_type=jax.ShapeDtypeStruct((_N, _D), jnp.float32),
        scratch_types=[
            pltpu.VMEM((_CH,), jnp.int32),
            pltpu.VMEM((_CH, _D), jnp.float32),
            pltpu.SemaphoreType.DMA,
        ],
    )(_gather_body)
    return f(tab, xi)


def _ll_body(llt_ref, x_ref, out_ref):
    i = pl.program_id(0)
    xs = x_ref[:, pl.ds(i * _LBW, _LBW)]
    m = lax.broadcasted_iota(jnp.int32, (_M, 1), 0)
    onehot = (xs == m).astype(jnp.float32)
    out_ref[...] = jnp.dot(llt_ref[...], onehot,
                           preferred_element_type=jnp.float32)


def _ll_matmul(llt, x2):
    return pl.pallas_call(
        _ll_body,
        grid=(_NLB,),
        in_specs=[
            pl.BlockSpec((_NG, _M), lambda i: (0, 0)),
            pl.BlockSpec((1, _NPAD), lambda i: (0, 0)),
        ],
        out_specs=pl.BlockSpec((_NG, _LBW), lambda i: (0, i)),
        out_shape=jax.ShapeDtypeStruct((_NG, _N), jnp.float32),
    )(llt, x2)


def kernel(x, B, Pi):
    xi = x.astype(jnp.int32)
    b2d = jnp.transpose(B, (1, 0, 2)).reshape(_M, _D)
    pi2d = jnp.broadcast_to(Pi.reshape(1, _D), (8, _D))
    tab, llt = _build_tables(b2d, pi2d)
    post_flat = _gather(tab, xi)
    xp = jnp.pad(xi, (0, _NPAD - _N)).reshape(1, _NPAD)
    ll_t = _ll_matmul(llt, xp)
    return jnp.transpose(ll_t, (1, 0)), post_flat.reshape(_N, _C, _NG)
